# Initial kernel scaffold; baseline (speedup 1.0000x reference)
import jax
import jax.numpy as jnp
from jax.experimental import pallas as pl


def kernel(x, edge_index, W1l, W1r, b1, W2l, W2r, b2, W3l, W3r, b3, Wlin, blin):
    raise NotImplementedError("write your pallas kernel here")


# speedup vs baseline: 4.7807x; 4.7807x over previous
import functools

import jax
import jax.numpy as jnp
from jax import lax
from jax.experimental import pallas as pl
from jax.experimental.pallas import tpu as pltpu
from jax.experimental.pallas import tpu_sc as plsc

N = 10000
E = 320000
D = 128

NC = 2
NS = 16
NW = NC * NS
EPT = E // NW
C = 80
NCHUNK = EPT // C

NP = 10240
RPT = NP // NS


def _zero_fill(buf, rows):
    def zfill(i, _):
        r = i // (D // 16)
        j = i % (D // 16)
        buf[r, pl.ds(j * 16, 16)] = jnp.zeros((16,), jnp.float32)
        return 0
    lax.fori_loop(0, rows * (D // 16), zfill, 0)


def _sc_agg_body(table, src, dst, agg_out, idx_s, idx_d, rows_v, agg_sh, sem):
    c = lax.axis_index("c")
    s = lax.axis_index("s")
    w = c * NS + s

    _zero_fill(rows_v, C)
    row0 = pl.multiple_of(s * RPT, RPT)
    for k in range(RPT // C):
        pltpu.sync_copy(rows_v, agg_sh.at[pl.ds(row0 + k * C, C)])
    plsc.subcore_barrier()

    base = pl.multiple_of(w * EPT, EPT)

    def body(i, _):
        off = pl.multiple_of(base + i * C, C)
        pltpu.sync_copy(src.at[pl.ds(off, C)], idx_s)
        pltpu.sync_copy(dst.at[pl.ds(off, C)], idx_d)
        pltpu.async_copy(table.at[idx_s], rows_v, sem).wait()
        pltpu.sync_copy(rows_v, agg_sh.at[idx_d], add=True)
        return 0
    lax.fori_loop(0, NCHUNK, body, 0)

    plsc.subcore_barrier()

    out_off = pl.multiple_of(c * NP + row0, RPT)
    for k in range(RPT // C):
        pltpu.sync_copy(agg_sh.at[pl.ds(row0 + k * C, C)], rows_v)
        pltpu.sync_copy(rows_v, agg_out.at[pl.ds(out_off + k * C, C)])


def _sc_deg_body(dst, deg_out, idx_d, ones_v, deg_sh, sem):
    c = lax.axis_index("c")
    s = lax.axis_index("s")
    w = c * NS + s

    _zero_fill(ones_v, C)
    row0 = pl.multiple_of(s * RPT, RPT)
    for k in range(RPT // C):
        pltpu.sync_copy(ones_v, deg_sh.at[pl.ds(row0 + k * C, C)])

    def ofill(i, _):
        r = i // (D // 16)
        j = i % (D // 16)
        ones_v[r, pl.ds(j * 16, 16)] = jnp.ones((16,), jnp.float32)
        return 0
    lax.fori_loop(0, C * (D // 16), ofill, 0)
    plsc.subcore_barrier()

    base = pl.multiple_of(w * EPT, EPT)

    def body(i, _):
        off = pl.multiple_of(base + i * C, C)
        pltpu.sync_copy(dst.at[pl.ds(off, C)], idx_d)
        pltpu.sync_copy(ones_v, deg_sh.at[idx_d], add=True)
        return 0
    lax.fori_loop(0, NCHUNK, body, 0)

    plsc.subcore_barrier()

    out_off = pl.multiple_of(c * NP + row0, RPT)
    for k in range(RPT // C):
        pltpu.sync_copy(deg_sh.at[pl.ds(row0 + k * C, C)], ones_v)
        pltpu.sync_copy(ones_v, deg_out.at[pl.ds(out_off + k * C, C)])


@functools.lru_cache(maxsize=None)
def _make_sc_agg():
    mesh = plsc.VectorSubcoreMesh(core_axis_name="c", subcore_axis_name="s")
    return pl.kernel(
        _sc_agg_body,
        out_type=jax.ShapeDtypeStruct((NC * NP, D), jnp.float32),
        scratch_types=[
            pltpu.VMEM((C,), jnp.int32),
            pltpu.VMEM((C,), jnp.int32),
            pltpu.VMEM((C, D), jnp.float32),
            pltpu.VMEM_SHARED((NP, D), jnp.float32),
            pltpu.SemaphoreType.DMA,
        ],
        mesh=mesh,
    )


@functools.lru_cache(maxsize=None)
def _make_sc_deg():
    mesh = plsc.VectorSubcoreMesh(core_axis_name="c", subcore_axis_name="s")
    return pl.kernel(
        _sc_deg_body,
        out_type=jax.ShapeDtypeStruct((NC * NP, D), jnp.float32),
        scratch_types=[
            pltpu.VMEM((C,), jnp.int32),
            pltpu.VMEM((C, D), jnp.float32),
            pltpu.VMEM_SHARED((NP, D), jnp.float32),
            pltpu.SemaphoreType.DMA,
        ],
        mesh=mesh,
    )


BN = 1000


def _tc_combine_body(P, dg, h, WlT, WrT, b, o, *, relu):
    d = dg[0, :, 0:1] + dg[1, :, 0:1]
    rdeg = 1.0 / jnp.maximum(d, 1.0)
    mean = (P[0] + P[1]) * rdeg
    acc = jnp.dot(mean, WlT[...], preferred_element_type=jnp.float32)
    acc += jnp.dot(h[...], WrT[...], preferred_element_type=jnp.float32)
    acc += b[...]
    o[...] = jnp.maximum(acc, 0.0) if relu else acc


def _tc_final_body(P, dg, h, WlT, WrT, b, WoT, bo, o):
    d = dg[0, :, 0:1] + dg[1, :, 0:1]
    rdeg = 1.0 / jnp.maximum(d, 1.0)
    mean = (P[0] + P[1]) * rdeg
    acc = jnp.dot(mean, WlT[...], preferred_element_type=jnp.float32)
    acc += jnp.dot(h[...], WrT[...], preferred_element_type=jnp.float32)
    acc += b[...]
    o[...] = (jnp.dot(acc, WoT[...], preferred_element_type=jnp.float32)
              + bo[...])


_P_SPEC = pl.BlockSpec((2, BN, D), lambda i: (0, i, 0))
_H_SPEC = pl.BlockSpec((BN, D), lambda i: (i, 0))
_W_SPEC = pl.BlockSpec((D, D), lambda i: (0, 0))
_B_SPEC = pl.BlockSpec((1, D), lambda i: (0, 0))


def _tc_combine(P, dg, h, WlT, WrT, b, relu):
    return pl.pallas_call(
        functools.partial(_tc_combine_body, relu=relu),
        grid=(N // BN,),
        in_specs=[_P_SPEC, _P_SPEC, _H_SPEC, _W_SPEC, _W_SPEC, _B_SPEC],
        out_specs=_H_SPEC,
        out_shape=jax.ShapeDtypeStruct((N, D), jnp.float32),
    )(P, dg, h, WlT, WrT, b)


def _tc_final(P, dg, h, WlT, WrT, b, WoT, bo):
    return pl.pallas_call(
        _tc_final_body,
        grid=(N // BN,),
        in_specs=[_P_SPEC, _P_SPEC, _H_SPEC, _W_SPEC, _W_SPEC, _B_SPEC,
                  _W_SPEC, _B_SPEC],
        out_specs=_H_SPEC,
        out_shape=jax.ShapeDtypeStruct((N, D), jnp.float32),
    )(P, dg, h, WlT, WrT, b, WoT, bo)


def kernel(x, edge_index, W1l, W1r, b1, W2l, W2r, b2, W3l, W3r, b3,
           Wlin, blin):
    src = edge_index[0]
    dst = edge_index[1]

    sc_agg = _make_sc_agg()
    DG = _make_sc_deg()(dst).reshape(NC, NP, D)

    P1 = sc_agg(x, src, dst).reshape(NC, NP, D)
    h1 = _tc_combine(P1, DG, x, W1l.T, W1r.T, b1.reshape(1, D), True)

    P2 = sc_agg(h1, src, dst).reshape(NC, NP, D)
    h2 = _tc_combine(P2, DG, h1, W2l.T, W2r.T, b2.reshape(1, D), True)

    P3 = sc_agg(h2, src, dst).reshape(NC, NP, D)
    out = _tc_final(P3, DG, h2, W3l.T, W3r.T, b3.reshape(1, D),
                    Wlin.T, blin.reshape(1, D))
    return out


# speedup vs baseline: 6.9068x; 1.4447x over previous
import functools

import jax
import jax.numpy as jnp
from jax import lax
from jax.experimental import pallas as pl
from jax.experimental.pallas import tpu as pltpu
from jax.experimental.pallas import tpu_sc as plsc

N = 10000
E = 320000
D = 128

NC = 2
NS = 16
NW = NC * NS
EPT = E // NW
C = 40
NCHUNK = EPT // C
PAIRS = NCHUNK // 2
DEG_BATCH = 10

NP = 10240
RPT = NP // NS
WB = RPT // C


def _fill(buf, rows, value):
    def body(i, _):
        r = i // (D // 16)
        j = i % (D // 16)
        buf[r, pl.ds(j * 16, 16)] = jnp.full((16,), value, jnp.float32)
        return 0
    lax.fori_loop(0, rows * (D // 16), body, 0)


def _sc_agg_body(table, src2, dst2, agg_out, idx_s2, idx_d2, rows2, agg_sh,
                 g0, g1, is0, is1):
    c = lax.axis_index("c")
    s = lax.axis_index("s")
    w = c * NS + s
    row0 = pl.multiple_of(s * RPT, RPT)
    gsems = (g0, g1)
    isems = (is0, is1)

    ebase = w * EPT

    def load_idx(i, b):
        off = pl.multiple_of(ebase + i * C, 8)
        pltpu.async_copy(src2.at[pl.ds(off, C)], idx_s2.at[b], isems[b])
        pltpu.async_copy(dst2.at[pl.ds(off, C)], idx_d2.at[b], isems[b])

    def drain_idx(b):
        pltpu.make_async_copy(src2.at[pl.ds(0, C)], idx_s2.at[b],
                              isems[b]).wait()
        pltpu.make_async_copy(src2.at[pl.ds(0, C)], idx_d2.at[b],
                              isems[b]).wait()

    def gather(i, b):
        pltpu.async_copy(table.at[idx_s2.at[b]], rows2.at[b], gsems[b])

    def drain_rows(b):
        pltpu.make_async_copy(table.at[pl.ds(0, C)], rows2.at[b],
                              gsems[b]).wait()

    def scatter(b):
        pltpu.sync_copy(rows2.at[b], agg_sh.at[idx_d2.at[b]], add=True)

    _fill(rows2.at[0], C, 0.0)
    for k in range(WB):
        pltpu.async_copy(rows2.at[0], agg_sh.at[pl.ds(row0 + k * C, C)], g0)
    load_idx(0, 0)
    load_idx(1, 1)
    for k in range(WB):
        drain_rows(0)
    plsc.subcore_barrier()

    drain_idx(0)
    gather(0, 0)

    def pair(t, _):
        i0 = t * 2
        drain_idx(1)
        gather(i0 + 1, 1)
        drain_rows(0)
        scatter(0)

        @pl.when(t + 1 < PAIRS)
        def _():
            load_idx(i0 + 2, 0)
        drain_rows(1)
        scatter(1)

        @pl.when(t + 1 < PAIRS)
        def _():
            drain_idx(0)
            gather(i0 + 2, 0)
            load_idx(i0 + 3, 1)
        return 0
    lax.fori_loop(0, PAIRS, pair, 0)

    plsc.subcore_barrier()

    out_off = pl.multiple_of(c * NP + row0, RPT)
    pltpu.async_copy(agg_sh.at[pl.ds(row0, C)], rows2.at[0], g0)
    for k in range(WB):
        b = k % 2
        if k + 1 < WB:
            pltpu.async_copy(agg_sh.at[pl.ds(row0 + (k + 1) * C, C)],
                             rows2.at[1 - b], gsems[1 - b])
        drain_rows(b)
        pltpu.sync_copy(rows2.at[b], agg_out.at[pl.ds(out_off + k * C, C)])


def _sc_deg_body(dst2, deg_out, idx_d3, ones_v, deg_sh, sem, isem):
    c = lax.axis_index("c")
    s = lax.axis_index("s")
    w = c * NS + s
    row0 = pl.multiple_of(s * RPT, RPT)
    NB = NCHUNK // DEG_BATCH

    def drain_scatters(n):
        for _ in range(n):
            pltpu.make_async_copy(deg_sh.at[pl.ds(0, C)], ones_v, sem).wait()

    ebase = w * EPT

    def load_batch(g, b):
        for k in range(DEG_BATCH):
            off = pl.multiple_of(ebase + (g * DEG_BATCH + k) * C, 8)
            pltpu.async_copy(dst2.at[pl.ds(off, C)],
                             idx_d3.at[b * DEG_BATCH + k], isem)

    def drain_batch():
        for _ in range(DEG_BATCH):
            pltpu.make_async_copy(dst2.at[pl.ds(0, C)], idx_d3.at[0],
                                  isem).wait()

    _fill(ones_v, C, 0.0)
    for k in range(WB):
        pltpu.async_copy(ones_v, deg_sh.at[pl.ds(row0 + k * C, C)], sem)
    load_batch(0, 0)
    drain_scatters(WB)
    drain_batch()
    _fill(ones_v, C, 1.0)
    plsc.subcore_barrier()

    def batch(g, _):
        b = lax.rem(g, 2)
        for k in range(DEG_BATCH):
            pltpu.async_copy(ones_v, deg_sh.at[idx_d3.at[b * DEG_BATCH + k]],
                             sem, add=True)

        @pl.when(g + 1 < NB)
        def _():
            load_batch(g + 1, 1 - b)
            drain_batch()
        drain_scatters(DEG_BATCH)
        return 0
    lax.fori_loop(0, NB, batch, 0)

    plsc.subcore_barrier()

    out_off = pl.multiple_of(c * NP + row0, RPT)
    for k in range(WB):
        pltpu.sync_copy(deg_sh.at[pl.ds(row0 + k * C, C)], ones_v)
        pltpu.sync_copy(ones_v, deg_out.at[pl.ds(out_off + k * C, C)])


@functools.lru_cache(maxsize=None)
def _make_sc_agg():
    mesh = plsc.VectorSubcoreMesh(core_axis_name="c", subcore_axis_name="s")
    return pl.kernel(
        _sc_agg_body,
        out_type=jax.ShapeDtypeStruct((NC * NP, D), jnp.float32),
        scratch_types=[
            pltpu.VMEM((2, C), jnp.int32),
            pltpu.VMEM((2, C), jnp.int32),
            pltpu.VMEM((2, C, D), jnp.float32),
            pltpu.VMEM_SHARED((NP, D), jnp.float32),
            pltpu.SemaphoreType.DMA,
            pltpu.SemaphoreType.DMA,
            pltpu.SemaphoreType.DMA,
            pltpu.SemaphoreType.DMA,
        ],
        mesh=mesh,
    )


@functools.lru_cache(maxsize=None)
def _make_sc_deg():
    mesh = plsc.VectorSubcoreMesh(core_axis_name="c", subcore_axis_name="s")
    return pl.kernel(
        _sc_deg_body,
        out_type=jax.ShapeDtypeStruct((NC * NP, D), jnp.float32),
        scratch_types=[
            pltpu.VMEM((2 * DEG_BATCH, C), jnp.int32),
            pltpu.VMEM((C, D), jnp.float32),
            pltpu.VMEM_SHARED((NP, D), jnp.float32),
            pltpu.SemaphoreType.DMA,
            pltpu.SemaphoreType.DMA,
        ],
        mesh=mesh,
    )


BN = 1000


def _tc_combine_body(P, dg, h, WlT, WrT, b, o, *, relu):
    d = dg[0, :, 0:1] + dg[1, :, 0:1]
    rdeg = 1.0 / jnp.maximum(d, 1.0)
    mean = (P[0] + P[1]) * rdeg
    acc = jnp.dot(mean, WlT[...], preferred_element_type=jnp.float32)
    acc += jnp.dot(h[...], WrT[...], preferred_element_type=jnp.float32)
    acc += b[...]
    o[...] = jnp.maximum(acc, 0.0) if relu else acc


def _tc_final_body(P, dg, h, WlT, WrT, b, WoT, bo, o):
    d = dg[0, :, 0:1] + dg[1, :, 0:1]
    rdeg = 1.0 / jnp.maximum(d, 1.0)
    mean = (P[0] + P[1]) * rdeg
    acc = jnp.dot(mean, WlT[...], preferred_element_type=jnp.float32)
    acc += jnp.dot(h[...], WrT[...], preferred_element_type=jnp.float32)
    acc += b[...]
    o[...] = (jnp.dot(acc, WoT[...], preferred_element_type=jnp.float32)
              + bo[...])


_P_SPEC = pl.BlockSpec((2, BN, D), lambda i: (0, i, 0))
_H_SPEC = pl.BlockSpec((BN, D), lambda i: (i, 0))
_W_SPEC = pl.BlockSpec((D, D), lambda i: (0, 0))
_B_SPEC = pl.BlockSpec((1, D), lambda i: (0, 0))


def _tc_combine(P, dg, h, WlT, WrT, b, relu):
    return pl.pallas_call(
        functools.partial(_tc_combine_body, relu=relu),
        grid=(N // BN,),
        in_specs=[_P_SPEC, _P_SPEC, _H_SPEC, _W_SPEC, _W_SPEC, _B_SPEC],
        out_specs=_H_SPEC,
        out_shape=jax.ShapeDtypeStruct((N, D), jnp.float32),
    )(P, dg, h, WlT, WrT, b)


def _tc_final(P, dg, h, WlT, WrT, b, WoT, bo):
    return pl.pallas_call(
        _tc_final_body,
        grid=(N // BN,),
        in_specs=[_P_SPEC, _P_SPEC, _H_SPEC, _W_SPEC, _W_SPEC, _B_SPEC,
                  _W_SPEC, _B_SPEC],
        out_specs=_H_SPEC,
        out_shape=jax.ShapeDtypeStruct((N, D), jnp.float32),
    )(P, dg, h, WlT, WrT, b, WoT, bo)


def kernel(x, edge_index, W1l, W1r, b1, W2l, W2r, b2, W3l, W3r, b3,
           Wlin, blin):
    src2 = edge_index[0]
    dst2 = edge_index[1]

    sc_agg = _make_sc_agg()
    DG = _make_sc_deg()(dst2).reshape(NC, NP, D)

    P1 = sc_agg(x, src2, dst2).reshape(NC, NP, D)
    h1 = _tc_combine(P1, DG, x, W1l.T, W1r.T, b1.reshape(1, D), True)

    P2 = sc_agg(h1, src2, dst2).reshape(NC, NP, D)
    h2 = _tc_combine(P2, DG, h1, W2l.T, W2r.T, b2.reshape(1, D), True)

    P3 = sc_agg(h2, src2, dst2).reshape(NC, NP, D)
    out = _tc_final(P3, DG, h2, W3l.T, W3r.T, b3.reshape(1, D),
                    Wlin.T, blin.reshape(1, D))
    return out


# speedup vs baseline: 9.4398x; 1.3667x over previous
import functools

import jax
import jax.numpy as jnp
from jax import lax
from jax.experimental import pallas as pl
from jax.experimental.pallas import tpu as pltpu
from jax.experimental.pallas import tpu_sc as plsc

N = 10000
E = 320000
D = 128

NC = 2
NS = 16
NW = NC * NS
EPT = E // NW
C = 40
NCHUNK = EPT // C
PAIRS = NCHUNK // 2
DEG_BATCH = 10

NP = 10240
RPT = NP // NS
WB = RPT // C


def _fill(buf, rows, value):
    def body(i, _):
        r = i // (D // 16)
        j = i % (D // 16)
        buf[r, pl.ds(j * 16, 16)] = jnp.full((16,), value, jnp.float32)
        return 0
    lax.fori_loop(0, rows * (D // 16), body, 0)


def _sc_agg_body(table, src2, dst2, agg_out, idx_s2, idx_d2, rows2, agg_sh,
                 g0, g1, s0, s1, isem):
    c = lax.axis_index("c")
    s = lax.axis_index("s")
    w = c * NS + s
    row0 = pl.multiple_of(s * RPT, RPT)
    gsems = (g0, g1)
    ssems = (s0, s1)

    ebase = w * EPT

    def load_idx(i, slot):
        off = pl.multiple_of(ebase + i * C, 8)
        pltpu.async_copy(src2.at[pl.ds(off, C)], idx_s2.at[slot], isem)
        pltpu.async_copy(dst2.at[pl.ds(off, C)], idx_d2.at[slot], isem)

    def drain_idx4():
        for _ in range(4):
            pltpu.make_async_copy(src2.at[pl.ds(0, C)], idx_s2.at[0],
                                  isem).wait()

    def gather(slot):
        pltpu.async_copy(table.at[idx_s2.at[slot]], rows2.at[slot],
                         gsems[slot % 2])

    def drain_g(b):
        pltpu.make_async_copy(table.at[pl.ds(0, C)], rows2.at[0],
                              gsems[b]).wait()

    def scatter(slot):
        pltpu.async_copy(rows2.at[slot], agg_sh.at[idx_d2.at[slot]],
                         ssems[slot % 2], add=True)

    def drain_s(b):
        pltpu.make_async_copy(table.at[pl.ds(0, C)], rows2.at[0],
                              ssems[b]).wait()

    _fill(rows2.at[0], C, 0.0)
    for k in range(WB):
        pltpu.async_copy(rows2.at[0], agg_sh.at[pl.ds(row0 + k * C, C)], g0)
    load_idx(0, 0)
    load_idx(1, 1)
    for k in range(WB):
        drain_g(0)
    plsc.subcore_barrier()

    drain_idx4()
    gather(0)
    gather(1)

    def pair_body(t, a, load_next, prime):
        o = 2 - a
        if prime:
            @pl.when(t > 0)
            def _():
                drain_s(0)
                drain_s(1)
        else:
            drain_s(0)
            drain_s(1)
        if load_next:
            load_idx(2 * t + 2, o)
            load_idx(2 * t + 3, o + 1)
        drain_g(0)
        scatter(a)
        drain_g(1)
        scatter(a + 1)
        if load_next:
            drain_idx4()
            gather(o)
            gather(o + 1)

    def quad(q, _):
        pair_body(2 * q, 0, True, True)
        pair_body(2 * q + 1, 2, True, False)
        return 0
    lax.fori_loop(0, (PAIRS - 1) // 2, quad, 0)
    pair_body(PAIRS - 1, 0, False, False)
    drain_s(0)
    drain_s(1)

    plsc.subcore_barrier()

    out_off = pl.multiple_of(c * NP + row0, RPT)
    pltpu.async_copy(agg_sh.at[pl.ds(row0, C)], rows2.at[0], g0)
    for k in range(WB):
        b = k % 2
        if k + 1 < WB:
            pltpu.async_copy(agg_sh.at[pl.ds(row0 + (k + 1) * C, C)],
                             rows2.at[1 - b], gsems[1 - b])
        drain_g(b)
        pltpu.sync_copy(rows2.at[b], agg_out.at[pl.ds(out_off + k * C, C)])


def _sc_deg_body(dst2, deg_out, idx_d3, ones_v, deg_sh, sem, isem):
    c = lax.axis_index("c")
    s = lax.axis_index("s")
    w = c * NS + s
    row0 = pl.multiple_of(s * RPT, RPT)
    NB = NCHUNK // DEG_BATCH

    def drain_scatters(n):
        for _ in range(n):
            pltpu.make_async_copy(deg_sh.at[pl.ds(0, C)], ones_v, sem).wait()

    ebase = w * EPT

    def load_batch(g, b):
        for k in range(DEG_BATCH):
            off = pl.multiple_of(ebase + (g * DEG_BATCH + k) * C, 8)
            pltpu.async_copy(dst2.at[pl.ds(off, C)],
                             idx_d3.at[b * DEG_BATCH + k], isem)

    def drain_batch():
        for _ in range(DEG_BATCH):
            pltpu.make_async_copy(dst2.at[pl.ds(0, C)], idx_d3.at[0],
                                  isem).wait()

    _fill(ones_v, C, 0.0)
    for k in range(WB):
        pltpu.async_copy(ones_v, deg_sh.at[pl.ds(row0 + k * C, C)], sem)
    load_batch(0, 0)
    drain_scatters(WB)
    drain_batch()
    _fill(ones_v, C, 1.0)
    plsc.subcore_barrier()

    def batch(g, _):
        b = lax.rem(g, 2)
        for k in range(DEG_BATCH):
            pltpu.async_copy(ones_v, deg_sh.at[idx_d3.at[b * DEG_BATCH + k]],
                             sem, add=True)

        @pl.when(g + 1 < NB)
        def _():
            load_batch(g + 1, 1 - b)
            drain_batch()
        drain_scatters(DEG_BATCH)
        return 0
    lax.fori_loop(0, NB, batch, 0)

    plsc.subcore_barrier()

    out_off = pl.multiple_of(c * NP + row0, RPT)
    for k in range(WB):
        pltpu.sync_copy(deg_sh.at[pl.ds(row0 + k * C, C)], ones_v)
        pltpu.sync_copy(ones_v, deg_out.at[pl.ds(out_off + k * C, C)])


@functools.lru_cache(maxsize=None)
def _make_sc_agg():
    mesh = plsc.VectorSubcoreMesh(core_axis_name="c", subcore_axis_name="s")
    return pl.kernel(
        _sc_agg_body,
        out_type=jax.ShapeDtypeStruct((NC * NP, D), jnp.float32),
        scratch_types=[
            pltpu.VMEM((4, C), jnp.int32),
            pltpu.VMEM((4, C), jnp.int32),
            pltpu.VMEM((4, C, D), jnp.float32),
            pltpu.VMEM_SHARED((NP, D), jnp.float32),
            pltpu.SemaphoreType.DMA,
            pltpu.SemaphoreType.DMA,
            pltpu.SemaphoreType.DMA,
            pltpu.SemaphoreType.DMA,
            pltpu.SemaphoreType.DMA,
        ],
        mesh=mesh,
    )


@functools.lru_cache(maxsize=None)
def _make_sc_deg():
    mesh = plsc.VectorSubcoreMesh(core_axis_name="c", subcore_axis_name="s")
    return pl.kernel(
        _sc_deg_body,
        out_type=jax.ShapeDtypeStruct((NC * NP, D), jnp.float32),
        scratch_types=[
            pltpu.VMEM((2 * DEG_BATCH, C), jnp.int32),
            pltpu.VMEM((C, D), jnp.float32),
            pltpu.VMEM_SHARED((NP, D), jnp.float32),
            pltpu.SemaphoreType.DMA,
            pltpu.SemaphoreType.DMA,
        ],
        mesh=mesh,
    )


BN = 1000


def _tc_combine_body(P, dg, h, WlT, WrT, b, o, *, relu):
    d = dg[0, :, 0:1] + dg[1, :, 0:1]
    rdeg = 1.0 / jnp.maximum(d, 1.0)
    mean = (P[0] + P[1]) * rdeg
    acc = jnp.dot(mean, WlT[...], preferred_element_type=jnp.float32)
    acc += jnp.dot(h[...], WrT[...], preferred_element_type=jnp.float32)
    acc += b[...]
    o[...] = jnp.maximum(acc, 0.0) if relu else acc


def _tc_final_body(P, dg, h, WlT, WrT, b, WoT, bo, o):
    d = dg[0, :, 0:1] + dg[1, :, 0:1]
    rdeg = 1.0 / jnp.maximum(d, 1.0)
    mean = (P[0] + P[1]) * rdeg
    acc = jnp.dot(mean, WlT[...], preferred_element_type=jnp.float32)
    acc += jnp.dot(h[...], WrT[...], preferred_element_type=jnp.float32)
    acc += b[...]
    o[...] = (jnp.dot(acc, WoT[...], preferred_element_type=jnp.float32)
              + bo[...])


_P_SPEC = pl.BlockSpec((2, BN, D), lambda i: (0, i, 0))
_H_SPEC = pl.BlockSpec((BN, D), lambda i: (i, 0))
_W_SPEC = pl.BlockSpec((D, D), lambda i: (0, 0))
_B_SPEC = pl.BlockSpec((1, D), lambda i: (0, 0))


def _tc_combine(P, dg, h, WlT, WrT, b, relu):
    return pl.pallas_call(
        functools.partial(_tc_combine_body, relu=relu),
        grid=(N // BN,),
        in_specs=[_P_SPEC, _P_SPEC, _H_SPEC, _W_SPEC, _W_SPEC, _B_SPEC],
        out_specs=_H_SPEC,
        out_shape=jax.ShapeDtypeStruct((N, D), jnp.float32),
    )(P, dg, h, WlT, WrT, b)


def _tc_final(P, dg, h, WlT, WrT, b, WoT, bo):
    return pl.pallas_call(
        _tc_final_body,
        grid=(N // BN,),
        in_specs=[_P_SPEC, _P_SPEC, _H_SPEC, _W_SPEC, _W_SPEC, _B_SPEC,
                  _W_SPEC, _B_SPEC],
        out_specs=_H_SPEC,
        out_shape=jax.ShapeDtypeStruct((N, D), jnp.float32),
    )(P, dg, h, WlT, WrT, b, WoT, bo)


def kernel(x, edge_index, W1l, W1r, b1, W2l, W2r, b2, W3l, W3r, b3,
           Wlin, blin):
    src2 = edge_index[0]
    dst2 = edge_index[1]

    sc_agg = _make_sc_agg()
    DG = _make_sc_deg()(dst2).reshape(NC, NP, D)

    P1 = sc_agg(x, src2, dst2).reshape(NC, NP, D)
    h1 = _tc_combine(P1, DG, x, W1l.T, W1r.T, b1.reshape(1, D), True)

    P2 = sc_agg(h1, src2, dst2).reshape(NC, NP, D)
    h2 = _tc_combine(P2, DG, h1, W2l.T, W2r.T, b2.reshape(1, D), True)

    P3 = sc_agg(h2, src2, dst2).reshape(NC, NP, D)
    out = _tc_final(P3, DG, h2, W3l.T, W3r.T, b3.reshape(1, D),
                    Wlin.T, blin.reshape(1, D))
    return out


# speedup vs baseline: 9.5573x; 1.0124x over previous
import functools

import jax
import jax.numpy as jnp
from jax import lax
from jax.experimental import pallas as pl
from jax.experimental.pallas import tpu as pltpu
from jax.experimental.pallas import tpu_sc as plsc

N = 10000
E = 320000
D = 128

NC = 2
NS = 16
NW = NC * NS
EPT = E // NW
C = 40
NCHUNK = EPT // C
PAIRS = NCHUNK // 2
DEG_BATCH = 25

NP = 10240
RPT = NP // NS
WB = RPT // C


def _fill(buf, rows, value):
    def body(i, _):
        r = i // (D // 16)
        j = i % (D // 16)
        buf[r, pl.ds(j * 16, 16)] = jnp.full((16,), value, jnp.float32)
        return 0
    lax.fori_loop(0, rows * (D // 16), body, 0)


def _sc_agg_body(table, src2, dst2, agg_out, idx_s2, idx_d2, rows2, agg_sh,
                 g0, g1, s0, s1, isem):
    c = lax.axis_index("c")
    s = lax.axis_index("s")
    w = c * NS + s
    row0 = pl.multiple_of(s * RPT, RPT)
    gsems = (g0, g1)
    ssems = (s0, s1)

    ebase = w * EPT

    def load_idx(i, slot):
        off = pl.multiple_of(ebase + i * C, 8)
        pltpu.async_copy(src2.at[pl.ds(off, C)], idx_s2.at[slot], isem)
        pltpu.async_copy(dst2.at[pl.ds(off, C)], idx_d2.at[slot], isem)

    def drain_idx4():
        for _ in range(4):
            pltpu.make_async_copy(src2.at[pl.ds(0, C)], idx_s2.at[0],
                                  isem).wait()

    def gather(slot):
        pltpu.async_copy(table.at[idx_s2.at[slot]], rows2.at[slot],
                         gsems[slot % 2])

    def drain_g(b):
        pltpu.make_async_copy(table.at[pl.ds(0, C)], rows2.at[0],
                              gsems[b]).wait()

    def scatter(slot):
        pltpu.async_copy(rows2.at[slot], agg_sh.at[idx_d2.at[slot]],
                         ssems[slot % 2], add=True)

    def drain_s(b):
        pltpu.make_async_copy(table.at[pl.ds(0, C)], rows2.at[0],
                              ssems[b]).wait()

    _fill(rows2.at[0], C, 0.0)
    for k in range(WB):
        pltpu.async_copy(rows2.at[0], agg_sh.at[pl.ds(row0 + k * C, C)], g0)
    load_idx(0, 0)
    load_idx(1, 1)
    for k in range(WB):
        drain_g(0)
    plsc.subcore_barrier()

    drain_idx4()
    gather(0)
    gather(1)

    def pair_body(t, a, load_next, prime):
        o = 2 - a
        if prime:
            @pl.when(t > 0)
            def _():
                drain_s(0)
                drain_s(1)
        else:
            drain_s(0)
            drain_s(1)
        if load_next:
            load_idx(2 * t + 2, o)
            load_idx(2 * t + 3, o + 1)
        drain_g(0)
        scatter(a)
        drain_g(1)
        scatter(a + 1)
        if load_next:
            drain_idx4()
            gather(o)
            gather(o + 1)

    def quad(q, _):
        pair_body(2 * q, 0, True, True)
        pair_body(2 * q + 1, 2, True, False)
        return 0
    lax.fori_loop(0, (PAIRS - 1) // 2, quad, 0)
    pair_body(PAIRS - 1, 0, False, False)
    drain_s(0)
    drain_s(1)

    plsc.subcore_barrier()

    out_off = pl.multiple_of(c * NP + row0, RPT)
    pltpu.async_copy(agg_sh.at[pl.ds(row0, C)], rows2.at[0], g0)
    for k in range(WB):
        b = k % 2
        if k + 1 < WB:
            pltpu.async_copy(agg_sh.at[pl.ds(row0 + (k + 1) * C, C)],
                             rows2.at[1 - b], gsems[1 - b])
        drain_g(b)
        pltpu.sync_copy(rows2.at[b], agg_out.at[pl.ds(out_off + k * C, C)])


def _sc_deg_body(dst2, deg_out, idx_d3, ones_v, deg_sh, sem, isem):
    c = lax.axis_index("c")
    s = lax.axis_index("s")
    w = c * NS + s
    row0 = pl.multiple_of(s * RPT, RPT)
    NB = NCHUNK // DEG_BATCH

    def drain_scatters(n):
        for _ in range(n):
            pltpu.make_async_copy(deg_sh.at[pl.ds(0, C)], ones_v, sem).wait()

    ebase = w * EPT

    def load_batch(g, b):
        for k in range(DEG_BATCH):
            off = pl.multiple_of(ebase + (g * DEG_BATCH + k) * C, 8)
            pltpu.async_copy(dst2.at[pl.ds(off, C)],
                             idx_d3.at[b * DEG_BATCH + k], isem)

    def drain_batch():
        for _ in range(DEG_BATCH):
            pltpu.make_async_copy(dst2.at[pl.ds(0, C)], idx_d3.at[0],
                                  isem).wait()

    _fill(ones_v, C, 0.0)
    for k in range(WB):
        pltpu.async_copy(ones_v, deg_sh.at[pl.ds(row0 + k * C, C)], sem)
    load_batch(0, 0)
    drain_scatters(WB)
    drain_batch()
    _fill(ones_v, C, 1.0)
    plsc.subcore_barrier()

    def batch(g, _):
        b = lax.rem(g, 2)
        for k in range(DEG_BATCH):
            pltpu.async_copy(ones_v, deg_sh.at[idx_d3.at[b * DEG_BATCH + k]],
                             sem, add=True)

        @pl.when(g + 1 < NB)
        def _():
            load_batch(g + 1, 1 - b)
            drain_batch()
        drain_scatters(DEG_BATCH)
        return 0
    lax.fori_loop(0, NB, batch, 0)

    plsc.subcore_barrier()

    out_off = pl.multiple_of(c * NP + row0, RPT)
    for k in range(WB):
        pltpu.sync_copy(deg_sh.at[pl.ds(row0 + k * C, C)], ones_v)
        pltpu.sync_copy(ones_v, deg_out.at[pl.ds(out_off + k * C, C)])


@functools.lru_cache(maxsize=None)
def _make_sc_agg():
    mesh = plsc.VectorSubcoreMesh(core_axis_name="c", subcore_axis_name="s")
    return pl.kernel(
        _sc_agg_body,
        out_type=jax.ShapeDtypeStruct((NC * NP, D), jnp.float32),
        scratch_types=[
            pltpu.VMEM((4, C), jnp.int32),
            pltpu.VMEM((4, C), jnp.int32),
            pltpu.VMEM((4, C, D), jnp.float32),
            pltpu.VMEM_SHARED((NP, D), jnp.float32),
            pltpu.SemaphoreType.DMA,
            pltpu.SemaphoreType.DMA,
            pltpu.SemaphoreType.DMA,
            pltpu.SemaphoreType.DMA,
            pltpu.SemaphoreType.DMA,
        ],
        mesh=mesh,
    )


@functools.lru_cache(maxsize=None)
def _make_sc_deg():
    mesh = plsc.VectorSubcoreMesh(core_axis_name="c", subcore_axis_name="s")
    return pl.kernel(
        _sc_deg_body,
        out_type=jax.ShapeDtypeStruct((NC * NP, D), jnp.float32),
        scratch_types=[
            pltpu.VMEM((2 * DEG_BATCH, C), jnp.int32),
            pltpu.VMEM((C, D), jnp.float32),
            pltpu.VMEM_SHARED((NP, D), jnp.float32),
            pltpu.SemaphoreType.DMA,
            pltpu.SemaphoreType.DMA,
        ],
        mesh=mesh,
    )


BN = 2000


def _tc_combine_body(P, dg, h, WlT, WrT, b, o, *, relu):
    d = dg[0, :, 0:1] + dg[1, :, 0:1]
    rdeg = 1.0 / jnp.maximum(d, 1.0)
    mean = (P[0] + P[1]) * rdeg
    acc = jnp.dot(mean, WlT[...], preferred_element_type=jnp.float32)
    acc += jnp.dot(h[...], WrT[...], preferred_element_type=jnp.float32)
    acc += b[...]
    o[...] = jnp.maximum(acc, 0.0) if relu else acc


def _tc_final_body(P, dg, h, WlT, WrT, b, WoT, bo, o):
    d = dg[0, :, 0:1] + dg[1, :, 0:1]
    rdeg = 1.0 / jnp.maximum(d, 1.0)
    mean = (P[0] + P[1]) * rdeg
    acc = jnp.dot(mean, WlT[...], preferred_element_type=jnp.float32)
    acc += jnp.dot(h[...], WrT[...], preferred_element_type=jnp.float32)
    acc += b[...]
    o[...] = (jnp.dot(acc, WoT[...], preferred_element_type=jnp.float32)
              + bo[...])


_P_SPEC = pl.BlockSpec((2, BN, D), lambda i: (0, i, 0))
_H_SPEC = pl.BlockSpec((BN, D), lambda i: (i, 0))
_W_SPEC = pl.BlockSpec((D, D), lambda i: (0, 0))
_B_SPEC = pl.BlockSpec((1, D), lambda i: (0, 0))


def _tc_combine(P, dg, h, WlT, WrT, b, relu):
    return pl.pallas_call(
        functools.partial(_tc_combine_body, relu=relu),
        grid=(N // BN,),
        in_specs=[_P_SPEC, _P_SPEC, _H_SPEC, _W_SPEC, _W_SPEC, _B_SPEC],
        out_specs=_H_SPEC,
        out_shape=jax.ShapeDtypeStruct((N, D), jnp.float32),
    )(P, dg, h, WlT, WrT, b)


def _tc_final(P, dg, h, WlT, WrT, b, WoT, bo):
    return pl.pallas_call(
        _tc_final_body,
        grid=(N // BN,),
        in_specs=[_P_SPEC, _P_SPEC, _H_SPEC, _W_SPEC, _W_SPEC, _B_SPEC,
                  _W_SPEC, _B_SPEC],
        out_specs=_H_SPEC,
        out_shape=jax.ShapeDtypeStruct((N, D), jnp.float32),
    )(P, dg, h, WlT, WrT, b, WoT, bo)


def kernel(x, edge_index, W1l, W1r, b1, W2l, W2r, b2, W3l, W3r, b3,
           Wlin, blin):
    src2 = edge_index[0]
    dst2 = edge_index[1]

    sc_agg = _make_sc_agg()
    DG = _make_sc_deg()(dst2).reshape(NC, NP, D)

    P1 = sc_agg(x, src2, dst2).reshape(NC, NP, D)
    h1 = _tc_combine(P1, DG, x, W1l.T, W1r.T, b1.reshape(1, D), True)

    P2 = sc_agg(h1, src2, dst2).reshape(NC, NP, D)
    h2 = _tc_combine(P2, DG, h1, W2l.T, W2r.T, b2.reshape(1, D), True)

    P3 = sc_agg(h2, src2, dst2).reshape(NC, NP, D)
    out = _tc_final(P3, DG, h2, W3l.T, W3r.T, b3.reshape(1, D),
                    Wlin.T, blin.reshape(1, D))
    return out
